# baseline (speedup 1.0000x reference)
import jax
import jax.numpy as jnp
from jax import lax
from jax.experimental import pallas as pl
from jax.experimental.pallas import tpu as pltpu
from jax.experimental.pallas import tpu_sc as plsc

N = 10000
E = 320000
D = 128
H = 8
C = 16
HID = 512
ED = 16

NC = 2
NS = 16
NW = NC * NS
K1 = 128
CHW1 = 80
K2 = 64
CHW2 = 158
EALLOC = (CHW1 + 2) * NW * K1
PW = 32
ROWS_PER_SUB = 640
TAIL_ROWS = N - 15 * ROWS_PER_SUB
TAIL_ROWS_Z = N + 8 - 15 * ROWS_PER_SUB

_F32 = jnp.float32
_I32 = jnp.int32

_SC_PARAMS = pltpu.CompilerParams(needs_layout_passes=False,
                                  use_tc_tiling_on_sc=False)


def _ln(r, g, b):
    mu = jnp.mean(r, axis=-1, keepdims=True)
    d = r - mu
    var = jnp.mean(d * d, axis=-1, keepdims=True)
    return d * lax.rsqrt(var + 1e-5) * g + b


def _dense_body(h_ref, w1_ref, w1b_ref, w2_ref, w2b_ref, g_ref, b_ref,
                ll_ref, llb_ref, lr_ref, lrb_ref, v_ref, xl_ref, xr_ref):
    hb = h_ref[...]
    a = lax.dot_general(hb, w1_ref[...], (((1,), (1,)), ((), ())),
                        preferred_element_type=_F32) + w1b_ref[...]
    a = jnp.maximum(a, 0.0)
    u = lax.dot_general(a, w2_ref[...], (((1,), (1,)), ((), ())),
                        preferred_element_type=_F32) + w2b_ref[...]
    v = _ln(hb + u, g_ref[...], b_ref[...])
    v_ref[...] = v
    xl_ref[...] = lax.dot_general(v, ll_ref[...], (((1,), (1,)), ((), ())),
                                  preferred_element_type=_F32) + llb_ref[...]
    xr_ref[...] = lax.dot_general(v, lr_ref[...], (((1,), (1,)), ((), ())),
                                  preferred_element_type=_F32) + lrb_ref[...]


def _tc_dense(h, w1_w, w1_b, w2_w, w2_b, ln_g, ln_b, ll_w, ll_b, lr_w, lr_b):
    R = 1000
    row = lambda i: (i, 0)
    fix = lambda i: (0, 0)
    out = [jax.ShapeDtypeStruct((N, D), _F32)] * 3
    return pl.pallas_call(
        _dense_body,
        grid=(N // R,),
        in_specs=[
            pl.BlockSpec((R, D), row),
            pl.BlockSpec((HID, D), fix),
            pl.BlockSpec((1, HID), fix),
            pl.BlockSpec((D, HID), fix),
            pl.BlockSpec((1, D), fix),
            pl.BlockSpec((1, D), fix),
            pl.BlockSpec((1, D), fix),
            pl.BlockSpec((D, D), fix),
            pl.BlockSpec((1, D), fix),
            pl.BlockSpec((D, D), fix),
            pl.BlockSpec((1, D), fix),
        ],
        out_specs=[pl.BlockSpec((R, D), row)] * 3,
        out_shape=out,
    )(h, w1_w, w1_b.reshape(1, HID), w2_w, w2_b.reshape(1, D),
      ln_g.reshape(1, D), ln_b.reshape(1, D), ll_w, ll_b.reshape(1, D),
      lr_w, lr_b.reshape(1, D))


def _ef_body(ea_ref, w_ref, out_ref):
    out_ref[...] = lax.dot_general(ea_ref[...], w_ref[...],
                                   (((1,), (1,)), ((), ())),
                                   preferred_element_type=_F32)


def _tc_ef(ea_p, lin_edge_w):
    R = 8192
    return pl.pallas_call(
        _ef_body,
        grid=(EALLOC // R,),
        in_specs=[pl.BlockSpec((R, ED), lambda i: (i, 0)),
                  pl.BlockSpec((D, ED), lambda i: (0, 0))],
        out_specs=pl.BlockSpec((R, D), lambda i: (i, 0)),
        out_shape=jax.ShapeDtypeStruct((EALLOC, D), _F32),
    )(ea_p, lin_edge_w)


def _spmem_rows(s, fn, tail):
    @pl.when(s < 15)
    def _():
        fn(s * ROWS_PER_SUB, ROWS_PER_SUB)

    @pl.when(s == 15)
    def _():
        fn(15 * ROWS_PER_SUB, tail)


def _sc_pass1(src_p, dst_p, xl, xr_p, ef_p, ea_p, att, z32):
    K = K1
    CHW = CHW1

    def body(src_h, dst_h, xl_h, xr_h, ef_h, ea_h, att_h, z32_h,
             ex_out, acc_p,
             acc_sh,
             sidx0, sidx1, didx0, didx1, dsc0, dsc1, xlb0, xlb1, xrb0, xrb1,
             efb0, efb1, pay0, pay1, exb, attbuf,
             semi0, semi1, semg0, semg1, sems0, sems1, semea0, semea1, semw):
        c = lax.axis_index("c")
        s = lax.axis_index("s")
        wid = s * NC + c
        sidx = (sidx0, sidx1)
        didx = (didx0, didx1)
        dsc = (dsc0, dsc1)
        xlb = (xlb0, xlb1)
        xrb = (xrb0, xrb1)
        efb = (efb0, efb1)
        pay = (pay0, pay1)
        semi = (semi0, semi1)
        semg = (semg0, semg1)
        sems = (sems0, sems1)
        semea = (semea0, semea1)

        pltpu.sync_copy(att_h, attbuf)

        zv = jnp.zeros((16,), _F32)
        cntv = jnp.where(lax.iota(_I32, 16) == 8, 1.0, 0.0).astype(_F32)

        def _initrow(e, _):
            for t in range(2):
                pay[t][e, pl.ds(0, 16)] = cntv
                pay[t][e, pl.ds(16, 16)] = zv
            return 0

        lax.fori_loop(0, K, _initrow, 0)

        def _z(start, n):
            pltpu.sync_copy(z32_h.at[pl.ds(start, n)],
                            acc_sh.at[pl.ds(start, n)])

        _spmem_rows(s, _z, TAIL_ROWS_Z)
        plsc.subcore_barrier()

        def _base(i):
            return (i * NW + wid) * K

        def issue_idx(t, i):
            b = _base(i)
            pltpu.async_copy(src_h.at[pl.ds(b, K)], sidx[t], semi[t])
            pltpu.async_copy(dst_h.at[pl.ds(b, K)], didx[t], semi[t])

        def wait_idx(t):
            pltpu.make_async_copy(src_h.at[pl.ds(0, K)], sidx[t], semi[t]).wait()
            pltpu.make_async_copy(dst_h.at[pl.ds(0, K)], didx[t], semi[t]).wait()

        def issue_gat(t, i):
            b = _base(i)
            pltpu.async_copy(xl_h.at[sidx[t]], xlb[t], semg[t])
            pltpu.async_copy(xr_h.at[didx[t]], xrb[t], semg[t])
            pltpu.async_copy(ef_h.at[pl.ds(b, K)], efb[t], semg[t])

        def wait_gat(t):
            pltpu.make_async_copy(xl_h.at[sidx[t]], xlb[t], semg[t]).wait()
            pltpu.make_async_copy(xr_h.at[didx[t]], xrb[t], semg[t]).wait()
            pltpu.make_async_copy(ef_h.at[pl.ds(0, K)], efb[t], semg[t]).wait()

        def wait_sc(t):
            pltpu.make_async_copy(
                pay[t], acc_sh.at[dsc[t]], sems[t]).wait()

        def wait_exw():
            pltpu.make_async_copy(
                exb, ex_out.at[pl.ds(0, K * H)], semw).wait()

        def compute_scatter(t, i):
            @pl.when(i >= 2)
            def _():
                wait_sc(t)

            pltpu.async_copy(ea_h.at[pl.ds(_base(i), K)],
                             pay[t].at[:, pl.ds(16, ED)], semea[t])

            @pl.when(i >= 1)
            def _():
                wait_exw()

            def _cpidx(q, _):
                dsc[t][pl.ds(q * 16, 16)] = didx[t][pl.ds(q * 16, 16)]
                return 0

            lax.fori_loop(0, K // 16, _cpidx, 0)
            issue_idx(t, i + 2)

            def _group(g, _):
                ev = lax.iota(_I32, 16) + g * 16
                ev8 = ev * H
                for h in range(H):
                    attrow = attbuf[h, :]
                    acc = jnp.zeros((16,), _F32)
                    for cc in range(C):
                        col = h * C + cc
                        cs = jnp.full((16,), col, _I32)
                        tv = (plsc.load_gather(xlb[t], [ev, cs])
                              + plsc.load_gather(xrb[t], [ev, cs])
                              + plsc.load_gather(efb[t], [ev, cs]))
                        tv = jnp.where(tv > 0, tv, tv * 0.2)
                        acc = acc + attrow[cc] * tv
                    exv = jnp.exp(acc)
                    plsc.store_scatter(
                        pay[t], [ev, jnp.full((16,), h, _I32)], exv)
                    plsc.store_scatter(exb, [ev8 + h], exv)
                return 0

            lax.fori_loop(0, K // 16, _group, 0)

            pltpu.make_async_copy(ea_h.at[pl.ds(0, K)],
                                  pay[t].at[:, pl.ds(16, ED)],
                                  semea[t]).wait()
            pltpu.async_copy(exb, ex_out.at[pl.ds(_base(i) * H, K * H)],
                             semw)
            pltpu.async_copy(pay[t], acc_sh.at[dsc[t]], sems[t], add=True)

        pltpu.sync_copy(src_h.at[pl.ds(_base(0), K)], sidx0)
        pltpu.sync_copy(dst_h.at[pl.ds(_base(0), K)], didx0)
        issue_gat(0, 0)
        issue_idx(1, 1)

        def _pair(j, _):
            i0 = 2 * j
            wait_idx(1)
            issue_gat(1, i0 + 1)
            wait_gat(0)
            compute_scatter(0, i0)

            wait_idx(0)
            issue_gat(0, i0 + 2)
            wait_gat(1)
            compute_scatter(1, i0 + 1)
            return 0

        lax.fori_loop(0, CHW // 2, _pair, 0)
        wait_gat(0)
        wait_idx(1)
        wait_sc(0)
        wait_sc(1)
        wait_exw()
        plsc.subcore_barrier()

        def _out(start, n):
            pltpu.sync_copy(acc_sh.at[pl.ds(start, n)],
                            acc_p.at[c, pl.ds(start, n)])

        _spmem_rows(s, _out, TAIL_ROWS)

    f = pl.kernel(
        body,
        out_type=[
            jax.ShapeDtypeStruct((EALLOC * H,), _F32),
            jax.ShapeDtypeStruct((NC, N, PW), _F32),
        ],
        mesh=plsc.VectorSubcoreMesh(core_axis_name="c", subcore_axis_name="s"),
        scratch_types=[
            pltpu.VMEM_SHARED((N + 8, PW), _F32),
            pltpu.VMEM((K,), _I32),
            pltpu.VMEM((K,), _I32),
            pltpu.VMEM((K,), _I32),
            pltpu.VMEM((K,), _I32),
            pltpu.VMEM((K,), _I32),
            pltpu.VMEM((K,), _I32),
            pltpu.VMEM((K, D), _F32),
            pltpu.VMEM((K, D), _F32),
            pltpu.VMEM((K, D), _F32),
            pltpu.VMEM((K, D), _F32),
            pltpu.VMEM((K, D), _F32),
            pltpu.VMEM((K, D), _F32),
            pltpu.VMEM((K, PW), _F32),
            pltpu.VMEM((K, PW), _F32),
            pltpu.VMEM((K * H,), _F32),
            pltpu.VMEM((H, C), _F32),
            pltpu.SemaphoreType.DMA,
            pltpu.SemaphoreType.DMA,
            pltpu.SemaphoreType.DMA,
            pltpu.SemaphoreType.DMA,
            pltpu.SemaphoreType.DMA,
            pltpu.SemaphoreType.DMA,
            pltpu.SemaphoreType.DMA,
            pltpu.SemaphoreType.DMA,
            pltpu.SemaphoreType.DMA,
        ],
        compiler_params=_SC_PARAMS,
    )
    return f(src_p, dst_p, xl, xr_p, ef_p, ea_p, att, z32)


def _sc_pass2(src_p, dst_p, xl, ex1d, invd16, z128):
    K = K2
    CHW = CHW2

    def body(src_h, dst_h, xl_h, ex_h, invd_h, z128_h,
             an_out, agg_p,
             agg_sh,
             sidx0, sidx1, didx0, didx1, dsc0, dsc1, xlb0, xlb1, invb0, invb1,
             exb0, exb1, anb0, anb1, msg0, msg1,
             semi0, semi1, semg0, semg1, sems0, sems1, semw0, semw1):
        c = lax.axis_index("c")
        s = lax.axis_index("s")
        wid = s * NC + c
        sidx = (sidx0, sidx1)
        didx = (didx0, didx1)
        dsc = (dsc0, dsc1)
        xlb = (xlb0, xlb1)
        invb = (invb0, invb1)
        exb = (exb0, exb1)
        anb = (anb0, anb1)
        msg = (msg0, msg1)
        semi = (semi0, semi1)
        semg = (semg0, semg1)
        sems = (sems0, sems1)
        semw = (semw0, semw1)

        def _z(start, n):
            pltpu.sync_copy(z128_h.at[pl.ds(start, n)],
                            agg_sh.at[pl.ds(start, n)])

        _spmem_rows(s, _z, TAIL_ROWS_Z)
        plsc.subcore_barrier()

        def _base(i):
            return (i * NW + wid) * K

        def issue_idx(t, i):
            b = _base(i)
            pltpu.async_copy(src_h.at[pl.ds(b, K)], sidx[t], semi[t])
            pltpu.async_copy(dst_h.at[pl.ds(b, K)], didx[t], semi[t])

        def wait_idx(t):
            pltpu.make_async_copy(src_h.at[pl.ds(0, K)], sidx[t], semi[t]).wait()
            pltpu.make_async_copy(dst_h.at[pl.ds(0, K)], didx[t], semi[t]).wait()

        def issue_gat(t, i):
            b = _base(i)
            pltpu.async_copy(xl_h.at[sidx[t]], xlb[t], semg[t])
            pltpu.async_copy(invd_h.at[didx[t]], invb[t], semg[t])
            pltpu.async_copy(ex_h.at[pl.ds(b * H, K * H)], exb[t], semg[t])

        def wait_gat(t):
            pltpu.make_async_copy(xl_h.at[sidx[t]], xlb[t], semg[t]).wait()
            pltpu.make_async_copy(invd_h.at[didx[t]], invb[t], semg[t]).wait()
            pltpu.make_async_copy(ex_h.at[pl.ds(0, K * H)], exb[t], semg[t]).wait()

        def wait_sc(t):
            pltpu.make_async_copy(
                anb[t], an_out.at[pl.ds(0, K * H)], semw[t]).wait()
            pltpu.make_async_copy(
                msg[t], agg_sh.at[dsc[t]], sems[t]).wait()

        def compute_scatter(t, i):
            @pl.when(i >= 2)
            def _():
                wait_sc(t)

            def _cpidx(q, _):
                dsc[t][pl.ds(q * 16, 16)] = didx[t][pl.ds(q * 16, 16)]
                return 0

            lax.fori_loop(0, K // 16, _cpidx, 0)
            issue_idx(t, i + 2)

            def _group(g, _):
                ev = lax.iota(_I32, 16) + g * 16
                ev8 = ev * H
                for h in range(H):
                    hs = jnp.full((16,), h, _I32)
                    exv = plsc.load_gather(exb[t], [ev8 + h])
                    iv = plsc.load_gather(invb[t], [ev, hs])
                    an = exv * iv
                    plsc.store_scatter(anb[t], [ev8 + h], an)
                    xvs = [plsc.load_gather(
                        xlb[t], [ev, jnp.full((16,), h * C + cc, _I32)])
                        for cc in range(C)]
                    mvs = [xv * an for xv in xvs]
                    for cc in range(C):
                        cs = jnp.full((16,), h * C + cc, _I32)
                        plsc.store_scatter(msg[t], [ev, cs], mvs[cc])
                return 0

            lax.fori_loop(0, K // 16, _group, 0)

            pltpu.async_copy(anb[t], an_out.at[pl.ds(_base(i) * H, K * H)],
                             semw[t])
            pltpu.async_copy(msg[t], agg_sh.at[dsc[t]], sems[t], add=True)

        pltpu.sync_copy(src_h.at[pl.ds(_base(0), K)], sidx0)
        pltpu.sync_copy(dst_h.at[pl.ds(_base(0), K)], didx0)
        issue_gat(0, 0)
        issue_idx(1, 1)

        def _pair(j, _):
            i0 = 2 * j
            wait_idx(1)
            issue_gat(1, i0 + 1)
            wait_gat(0)
            compute_scatter(0, i0)

            wait_idx(0)
            issue_gat(0, i0 + 2)
            wait_gat(1)
            compute_scatter(1, i0 + 1)
            return 0

        lax.fori_loop(0, CHW // 2, _pair, 0)
        wait_gat(0)
        wait_idx(1)
        wait_sc(0)
        wait_sc(1)
        plsc.subcore_barrier()

        def _out(start, n):
            pltpu.sync_copy(agg_sh.at[pl.ds(start, n)],
                            agg_p.at[c, pl.ds(start, n)])

        _spmem_rows(s, _out, TAIL_ROWS)

    f = pl.kernel(
        body,
        out_type=[
            jax.ShapeDtypeStruct((EALLOC * H,), _F32),
            jax.ShapeDtypeStruct((NC, N, D), _F32),
        ],
        mesh=plsc.VectorSubcoreMesh(core_axis_name="c", subcore_axis_name="s"),
        scratch_types=[
            pltpu.VMEM_SHARED((N + 8, D), _F32),
            pltpu.VMEM((K,), _I32),
            pltpu.VMEM((K,), _I32),
            pltpu.VMEM((K,), _I32),
            pltpu.VMEM((K,), _I32),
            pltpu.VMEM((K,), _I32),
            pltpu.VMEM((K,), _I32),
            pltpu.VMEM((K, D), _F32),
            pltpu.VMEM((K, D), _F32),
            pltpu.VMEM((K, ED), _F32),
            pltpu.VMEM((K, ED), _F32),
            pltpu.VMEM((K * H,), _F32),
            pltpu.VMEM((K * H,), _F32),
            pltpu.VMEM((K * H,), _F32),
            pltpu.VMEM((K * H,), _F32),
            pltpu.VMEM((K, D), _F32),
            pltpu.VMEM((K, D), _F32),
            pltpu.SemaphoreType.DMA,
            pltpu.SemaphoreType.DMA,
            pltpu.SemaphoreType.DMA,
            pltpu.SemaphoreType.DMA,
            pltpu.SemaphoreType.DMA,
            pltpu.SemaphoreType.DMA,
            pltpu.SemaphoreType.DMA,
            pltpu.SemaphoreType.DMA,
        ],
        compiler_params=_SC_PARAMS,
    )
    return f(src_p, dst_p, xl, ex1d, invd16, z128)


def _mid_body(acc_ref, xl_ref, xr_ref, lew_ref, amat_ref,
              inv_ref, anl_ref):
    acc = acc_ref[0] + acc_ref[1]
    asum = acc[:, 16:32]
    cnt = acc[:, 8:9]
    loop_attr = asum / jnp.maximum(cnt, 1.0)
    efl = lax.dot_general(loop_attr, lew_ref[...], (((1,), (1,)), ((), ())),
                          preferred_element_type=_F32)
    t = xl_ref[...] + xr_ref[...] + efl
    t = jnp.where(t > 0, t, t * 0.2)
    alpha = lax.dot_general(t, amat_ref[...], (((1,), (0,)), ((), ())),
                            preferred_element_type=_F32)
    exl = jnp.exp(alpha)
    denom = acc[:, 0:H] + exl
    inv = 1.0 / (denom + 1e-16)
    inv_ref[...] = inv
    anl_ref[...] = exl * inv


def _tc_mid(acc_p, xl, xr, lin_edge_w, amat):
    R = 1000
    row3 = lambda i: (0, i, 0)
    row = lambda i: (i, 0)
    fix = lambda i: (0, 0)
    return pl.pallas_call(
        _mid_body,
        grid=(N // R,),
        in_specs=[
            pl.BlockSpec((NC, R, PW), row3),
            pl.BlockSpec((R, D), row),
            pl.BlockSpec((R, D), row),
            pl.BlockSpec((D, ED), fix),
            pl.BlockSpec((D, H), fix),
        ],
        out_specs=[pl.BlockSpec((R, H), row)] * 2,
        out_shape=[jax.ShapeDtypeStruct((N, H), _F32)] * 2,
    )(acc_p, xl, xr, lin_edge_w, amat)


def _out_body(agg_ref, anl_ref, xl_ref, v_ref, bias_ref, g_ref, b_ref,
              bmat_ref, out_ref):
    anb = lax.dot_general(anl_ref[...], bmat_ref[...], (((1,), (0,)), ((), ())),
                          preferred_element_type=_F32)
    agg = agg_ref[0] + agg_ref[1] + xl_ref[...] * anb + bias_ref[...]
    out_ref[...] = _ln(agg + v_ref[...], g_ref[...], b_ref[...])


def _tc_out(agg_p, anl, xl, v, conv_bias, ln_g, ln_b, bmat):
    R = 1000
    row3 = lambda i: (0, i, 0)
    row = lambda i: (i, 0)
    fix = lambda i: (0, 0)
    return pl.pallas_call(
        _out_body,
        grid=(N // R,),
        in_specs=[
            pl.BlockSpec((NC, R, D), row3),
            pl.BlockSpec((R, H), row),
            pl.BlockSpec((R, D), row),
            pl.BlockSpec((R, D), row),
            pl.BlockSpec((1, D), fix),
            pl.BlockSpec((1, D), fix),
            pl.BlockSpec((1, D), fix),
            pl.BlockSpec((H, D), fix),
        ],
        out_specs=pl.BlockSpec((R, D), row),
        out_shape=jax.ShapeDtypeStruct((N, D), _F32),
    )(agg_p, anl, xl, v, conv_bias.reshape(1, D), ln_g.reshape(1, D),
      ln_b.reshape(1, D), bmat)


def kernel(h, edge_index, edge_attr, w1_w, w1_b, w2_w, w2_b, ln_g, ln_b,
           lin_l_w, lin_l_b, lin_r_w, lin_r_b, lin_edge_w, att, conv_bias):
    src = edge_index[0].astype(_I32)
    dst = edge_index[1].astype(_I32)
    pad = EALLOC - E
    src_p = jnp.concatenate([src, jnp.zeros((pad,), _I32)])
    dst_p = jnp.concatenate([dst, jnp.full((pad,), N, _I32)])
    ea_p = jnp.concatenate([edge_attr, jnp.zeros((pad, ED), _F32)])

    v, xl, xr = _tc_dense(h, w1_w, w1_b, w2_w, w2_b, ln_g, ln_b,
                          lin_l_w, lin_l_b, lin_r_w, lin_r_b)
    xr_p = jnp.concatenate([xr, jnp.zeros((8, D), _F32)])
    ef = _tc_ef(ea_p, lin_edge_w)

    z32 = jnp.zeros((N + 8, PW), _F32)
    z128 = jnp.zeros((N + 8, D), _F32)
    ex1d, acc_p = _sc_pass1(src_p, dst_p, xl, xr_p, ef, ea_p, att, z32)

    amat = jnp.zeros((D, H), _F32).at[
        jnp.arange(D), jnp.arange(D) // C].set(att.reshape(-1))
    invd, anl = _tc_mid(acc_p, xl, xr, lin_edge_w, amat)

    invd16 = jnp.concatenate(
        [jnp.concatenate([invd, jnp.zeros((N, ED - H), _F32)], axis=1),
         jnp.zeros((8, ED), _F32)], axis=0)
    an1d, agg_p = _sc_pass2(src_p, dst_p, xl, ex1d, invd16, z128)
    an_edges = an1d[:E * H].reshape(E, H)

    bmat = jnp.kron(jnp.eye(H, dtype=_F32), jnp.ones((1, C), _F32))
    output = _tc_out(agg_p, anl, xl, v, conv_bias, ln_g, ln_b, bmat)

    alpha_n = jnp.concatenate([an_edges, anl], axis=0)
    return (output, alpha_n)


# speedup vs baseline: 1.0661x; 1.0661x over previous
import jax
import jax.numpy as jnp
from jax import lax
from jax.experimental import pallas as pl
from jax.experimental.pallas import tpu as pltpu
from jax.experimental.pallas import tpu_sc as plsc

N = 10000
E = 320000
D = 128
H = 8
C = 16
HID = 512
ED = 16

NC = 2
NS = 16
NW = NC * NS
K1 = 64
CHW1 = 158
K2 = 64
CHW2 = 158
EALLOC = (CHW1 + 2) * NW * K1
PW = 32
ROWS_PER_SUB = 640
TAIL_ROWS = N - 15 * ROWS_PER_SUB
TAIL_ROWS_Z = N + 8 - 15 * ROWS_PER_SUB

_F32 = jnp.float32
_I32 = jnp.int32

_SC_PARAMS = pltpu.CompilerParams(needs_layout_passes=False,
                                  use_tc_tiling_on_sc=False)


def _ln(r, g, b):
    mu = jnp.mean(r, axis=-1, keepdims=True)
    d = r - mu
    var = jnp.mean(d * d, axis=-1, keepdims=True)
    return d * lax.rsqrt(var + 1e-5) * g + b


def _dense_body(h_ref, w1_ref, w1b_ref, w2_ref, w2b_ref, g_ref, b_ref,
                ll_ref, llb_ref, lr_ref, lrb_ref, v_ref, xl_ref, xr_ref):
    hb = h_ref[...]
    a = lax.dot_general(hb, w1_ref[...], (((1,), (1,)), ((), ())),
                        preferred_element_type=_F32) + w1b_ref[...]
    a = jnp.maximum(a, 0.0)
    u = lax.dot_general(a, w2_ref[...], (((1,), (1,)), ((), ())),
                        preferred_element_type=_F32) + w2b_ref[...]
    v = _ln(hb + u, g_ref[...], b_ref[...])
    v_ref[...] = v
    xl_ref[...] = lax.dot_general(v, ll_ref[...], (((1,), (1,)), ((), ())),
                                  preferred_element_type=_F32) + llb_ref[...]
    xr_ref[...] = lax.dot_general(v, lr_ref[...], (((1,), (1,)), ((), ())),
                                  preferred_element_type=_F32) + lrb_ref[...]


def _tc_dense(h, w1_w, w1_b, w2_w, w2_b, ln_g, ln_b, ll_w, ll_b, lr_w, lr_b):
    R = 1000
    row = lambda i: (i, 0)
    fix = lambda i: (0, 0)
    out = [jax.ShapeDtypeStruct((N, D), _F32)] * 3
    return pl.pallas_call(
        _dense_body,
        grid=(N // R,),
        in_specs=[
            pl.BlockSpec((R, D), row),
            pl.BlockSpec((HID, D), fix),
            pl.BlockSpec((1, HID), fix),
            pl.BlockSpec((D, HID), fix),
            pl.BlockSpec((1, D), fix),
            pl.BlockSpec((1, D), fix),
            pl.BlockSpec((1, D), fix),
            pl.BlockSpec((D, D), fix),
            pl.BlockSpec((1, D), fix),
            pl.BlockSpec((D, D), fix),
            pl.BlockSpec((1, D), fix),
        ],
        out_specs=[pl.BlockSpec((R, D), row)] * 3,
        out_shape=out,
    )(h, w1_w, w1_b.reshape(1, HID), w2_w, w2_b.reshape(1, D),
      ln_g.reshape(1, D), ln_b.reshape(1, D), ll_w, ll_b.reshape(1, D),
      lr_w, lr_b.reshape(1, D))


def _ef_body(ea_ref, w_ref, out_ref):
    out_ref[...] = lax.dot_general(ea_ref[...], w_ref[...],
                                   (((1,), (1,)), ((), ())),
                                   preferred_element_type=_F32)


def _tc_ef(ea_p, lin_edge_w):
    R = 8192
    return pl.pallas_call(
        _ef_body,
        grid=(EALLOC // R,),
        in_specs=[pl.BlockSpec((R, ED), lambda i: (i, 0)),
                  pl.BlockSpec((D, ED), lambda i: (0, 0))],
        out_specs=pl.BlockSpec((R, D), lambda i: (i, 0)),
        out_shape=jax.ShapeDtypeStruct((EALLOC, D), _F32),
    )(ea_p, lin_edge_w)


def _spmem_rows(s, fn, tail):
    @pl.when(s < 15)
    def _():
        fn(s * ROWS_PER_SUB, ROWS_PER_SUB)

    @pl.when(s == 15)
    def _():
        fn(15 * ROWS_PER_SUB, tail)


def _sc_pass1(src_p, dst_p, xl, xr_p, ef_p, ea_p, att, z32):
    K = K1
    CHW = CHW1

    def body(src_h, dst_h, xl_h, xr_h, ef_h, ea_h, att_h, z32_h,
             ex_out, acc_p,
             acc_sh,
             sidx0, sidx1, didx0, didx1, dsc0, dsc1, xlb0, xlb1, xrb0, xrb1,
             efb0, efb1, pay0, pay1, exb0, exb1, attbuf,
             semi0, semi1, semg0, semg1, sems0, sems1, semea0, semea1,
             semw0, semw1):
        c = lax.axis_index("c")
        s = lax.axis_index("s")
        wid = s * NC + c
        sidx = (sidx0, sidx1)
        didx = (didx0, didx1)
        dsc = (dsc0, dsc1)
        xlb = (xlb0, xlb1)
        xrb = (xrb0, xrb1)
        efb = (efb0, efb1)
        pay = (pay0, pay1)
        exb = (exb0, exb1)
        semi = (semi0, semi1)
        semg = (semg0, semg1)
        sems = (sems0, sems1)
        semea = (semea0, semea1)
        semw = (semw0, semw1)

        pltpu.sync_copy(att_h, attbuf)

        zv = jnp.zeros((16,), _F32)
        cntv = jnp.where(lax.iota(_I32, 16) == 8, 1.0, 0.0).astype(_F32)

        def _initrow(e, _):
            for t in range(2):
                pay[t][e, pl.ds(0, 16)] = cntv
                pay[t][e, pl.ds(16, 16)] = zv
            return 0

        lax.fori_loop(0, K, _initrow, 0)

        def _z(start, n):
            pltpu.sync_copy(z32_h.at[pl.ds(start, n)],
                            acc_sh.at[pl.ds(start, n)])

        _spmem_rows(s, _z, TAIL_ROWS_Z)
        plsc.subcore_barrier()

        def _base(i):
            return (i * NW + wid) * K

        def issue_idx(t, i):
            b = _base(i)
            pltpu.async_copy(src_h.at[pl.ds(b, K)], sidx[t], semi[t])
            pltpu.async_copy(dst_h.at[pl.ds(b, K)], didx[t], semi[t])

        def wait_idx(t):
            pltpu.make_async_copy(src_h.at[pl.ds(0, K)], sidx[t], semi[t]).wait()
            pltpu.make_async_copy(dst_h.at[pl.ds(0, K)], didx[t], semi[t]).wait()

        def issue_gat(t, i):
            b = _base(i)
            pltpu.async_copy(xl_h.at[sidx[t]], xlb[t], semg[t])
            pltpu.async_copy(xr_h.at[didx[t]], xrb[t], semg[t])
            pltpu.async_copy(ef_h.at[pl.ds(b, K)], efb[t], semg[t])

        def wait_gat(t):
            pltpu.make_async_copy(xl_h.at[sidx[t]], xlb[t], semg[t]).wait()
            pltpu.make_async_copy(xr_h.at[didx[t]], xrb[t], semg[t]).wait()
            pltpu.make_async_copy(ef_h.at[pl.ds(0, K)], efb[t], semg[t]).wait()

        def wait_sc(t):
            pltpu.make_async_copy(
                exb[t], ex_out.at[pl.ds(0, K * H)], semw[t]).wait()
            pltpu.make_async_copy(
                pay[t], acc_sh.at[dsc[t]], sems[t]).wait()

        def compute_scatter(t, i):
            @pl.when(i >= 2)
            def _():
                wait_sc(t)

            pltpu.async_copy(ea_h.at[pl.ds(_base(i), K)],
                             pay[t].at[:, pl.ds(16, ED)], semea[t])

            def _cpidx(q, _):
                dsc[t][pl.ds(q * 16, 16)] = didx[t][pl.ds(q * 16, 16)]
                return 0

            lax.fori_loop(0, K // 16, _cpidx, 0)
            issue_idx(t, i + 2)

            def _group(g, _):
                ev = lax.iota(_I32, 16) + g * 16
                ev8 = ev * H
                for h in range(H):
                    attrow = attbuf[h, :]
                    acc = jnp.zeros((16,), _F32)
                    for cc in range(C):
                        col = h * C + cc
                        cs = jnp.full((16,), col, _I32)
                        tv = (plsc.load_gather(xlb[t], [ev, cs])
                              + plsc.load_gather(xrb[t], [ev, cs])
                              + plsc.load_gather(efb[t], [ev, cs]))
                        tv = jnp.where(tv > 0, tv, tv * 0.2)
                        acc = acc + attrow[cc] * tv
                    exv = jnp.exp(acc)
                    plsc.store_scatter(
                        pay[t], [ev, jnp.full((16,), h, _I32)], exv)
                    plsc.store_scatter(exb[t], [ev8 + h], exv)
                return 0

            lax.fori_loop(0, K // 16, _group, 0)

            pltpu.make_async_copy(ea_h.at[pl.ds(0, K)],
                                  pay[t].at[:, pl.ds(16, ED)],
                                  semea[t]).wait()
            pltpu.async_copy(exb[t], ex_out.at[pl.ds(_base(i) * H, K * H)],
                             semw[t])
            pltpu.async_copy(pay[t], acc_sh.at[dsc[t]], sems[t], add=True)

        pltpu.sync_copy(src_h.at[pl.ds(_base(0), K)], sidx0)
        pltpu.sync_copy(dst_h.at[pl.ds(_base(0), K)], didx0)
        issue_gat(0, 0)
        issue_idx(1, 1)

        def _pair(j, _):
            i0 = 2 * j
            wait_idx(1)
            issue_gat(1, i0 + 1)
            wait_gat(0)
            compute_scatter(0, i0)

            wait_idx(0)
            issue_gat(0, i0 + 2)
            wait_gat(1)
            compute_scatter(1, i0 + 1)
            return 0

        lax.fori_loop(0, CHW // 2, _pair, 0)
        wait_gat(0)
        wait_idx(1)
        wait_sc(0)
        wait_sc(1)
        plsc.subcore_barrier()

        def _out(start, n):
            pltpu.sync_copy(acc_sh.at[pl.ds(start, n)],
                            acc_p.at[c, pl.ds(start, n)])

        _spmem_rows(s, _out, TAIL_ROWS)

    f = pl.kernel(
        body,
        out_type=[
            jax.ShapeDtypeStruct((EALLOC * H,), _F32),
            jax.ShapeDtypeStruct((NC, N, PW), _F32),
        ],
        mesh=plsc.VectorSubcoreMesh(core_axis_name="c", subcore_axis_name="s"),
        scratch_types=[
            pltpu.VMEM_SHARED((N + 8, PW), _F32),
            pltpu.VMEM((K,), _I32),
            pltpu.VMEM((K,), _I32),
            pltpu.VMEM((K,), _I32),
            pltpu.VMEM((K,), _I32),
            pltpu.VMEM((K,), _I32),
            pltpu.VMEM((K,), _I32),
            pltpu.VMEM((K, D), _F32),
            pltpu.VMEM((K, D), _F32),
            pltpu.VMEM((K, D), _F32),
            pltpu.VMEM((K, D), _F32),
            pltpu.VMEM((K, D), _F32),
            pltpu.VMEM((K, D), _F32),
            pltpu.VMEM((K, PW), _F32),
            pltpu.VMEM((K, PW), _F32),
            pltpu.VMEM((K * H,), _F32),
            pltpu.VMEM((K * H,), _F32),
            pltpu.VMEM((H, C), _F32),
            pltpu.SemaphoreType.DMA,
            pltpu.SemaphoreType.DMA,
            pltpu.SemaphoreType.DMA,
            pltpu.SemaphoreType.DMA,
            pltpu.SemaphoreType.DMA,
            pltpu.SemaphoreType.DMA,
            pltpu.SemaphoreType.DMA,
            pltpu.SemaphoreType.DMA,
            pltpu.SemaphoreType.DMA,
            pltpu.SemaphoreType.DMA,
        ],
        compiler_params=_SC_PARAMS,
    )
    return f(src_p, dst_p, xl, xr_p, ef_p, ea_p, att, z32)


def _sc_pass2(src_p, dst_p, xl, ex1d, invd16, z128):
    K = K2
    CHW = CHW2

    def body(src_h, dst_h, xl_h, ex_h, invd_h, z128_h,
             an_out, agg_p,
             agg_sh,
             sidx0, sidx1, didx0, didx1, dsc0, dsc1, xlb0, xlb1, invb0, invb1,
             exb0, exb1, anb0, anb1, msg0, msg1,
             semi0, semi1, semg0, semg1, sems0, sems1, semw0, semw1):
        c = lax.axis_index("c")
        s = lax.axis_index("s")
        wid = s * NC + c
        sidx = (sidx0, sidx1)
        didx = (didx0, didx1)
        dsc = (dsc0, dsc1)
        xlb = (xlb0, xlb1)
        invb = (invb0, invb1)
        exb = (exb0, exb1)
        anb = (anb0, anb1)
        msg = (msg0, msg1)
        semi = (semi0, semi1)
        semg = (semg0, semg1)
        sems = (sems0, sems1)
        semw = (semw0, semw1)

        def _z(start, n):
            pltpu.sync_copy(z128_h.at[pl.ds(start, n)],
                            agg_sh.at[pl.ds(start, n)])

        _spmem_rows(s, _z, TAIL_ROWS_Z)
        plsc.subcore_barrier()

        def _base(i):
            return (i * NW + wid) * K

        def issue_idx(t, i):
            b = _base(i)
            pltpu.async_copy(src_h.at[pl.ds(b, K)], sidx[t], semi[t])
            pltpu.async_copy(dst_h.at[pl.ds(b, K)], didx[t], semi[t])

        def wait_idx(t):
            pltpu.make_async_copy(src_h.at[pl.ds(0, K)], sidx[t], semi[t]).wait()
            pltpu.make_async_copy(dst_h.at[pl.ds(0, K)], didx[t], semi[t]).wait()

        def issue_gat(t, i):
            b = _base(i)
            pltpu.async_copy(xl_h.at[sidx[t]], xlb[t], semg[t])
            pltpu.async_copy(invd_h.at[didx[t]], invb[t], semg[t])
            pltpu.async_copy(ex_h.at[pl.ds(b * H, K * H)], exb[t], semg[t])

        def wait_gat(t):
            pltpu.make_async_copy(xl_h.at[sidx[t]], xlb[t], semg[t]).wait()
            pltpu.make_async_copy(invd_h.at[didx[t]], invb[t], semg[t]).wait()
            pltpu.make_async_copy(ex_h.at[pl.ds(0, K * H)], exb[t], semg[t]).wait()

        def wait_sc(t):
            pltpu.make_async_copy(
                anb[t], an_out.at[pl.ds(0, K * H)], semw[t]).wait()
            pltpu.make_async_copy(
                msg[t], agg_sh.at[dsc[t]], sems[t]).wait()

        def compute_scatter(t, i):
            @pl.when(i >= 2)
            def _():
                wait_sc(t)

            def _cpidx(q, _):
                dsc[t][pl.ds(q * 16, 16)] = didx[t][pl.ds(q * 16, 16)]
                return 0

            lax.fori_loop(0, K // 16, _cpidx, 0)
            issue_idx(t, i + 2)

            def _group(g, _):
                ev = lax.iota(_I32, 16) + g * 16
                ev8 = ev * H
                for h in range(H):
                    hs = jnp.full((16,), h, _I32)
                    exv = plsc.load_gather(exb[t], [ev8 + h])
                    iv = plsc.load_gather(invb[t], [ev, hs])
                    an = exv * iv
                    plsc.store_scatter(anb[t], [ev8 + h], an)
                    xvs = [plsc.load_gather(
                        xlb[t], [ev, jnp.full((16,), h * C + cc, _I32)])
                        for cc in range(C)]
                    mvs = [xv * an for xv in xvs]
                    for cc in range(C):
                        cs = jnp.full((16,), h * C + cc, _I32)
                        plsc.store_scatter(msg[t], [ev, cs], mvs[cc])
                return 0

            lax.fori_loop(0, K // 16, _group, 0)

            pltpu.async_copy(anb[t], an_out.at[pl.ds(_base(i) * H, K * H)],
                             semw[t])
            pltpu.async_copy(msg[t], agg_sh.at[dsc[t]], sems[t], add=True)

        pltpu.sync_copy(src_h.at[pl.ds(_base(0), K)], sidx0)
        pltpu.sync_copy(dst_h.at[pl.ds(_base(0), K)], didx0)
        issue_gat(0, 0)
        issue_idx(1, 1)

        def _pair(j, _):
            i0 = 2 * j
            wait_idx(1)
            issue_gat(1, i0 + 1)
            wait_gat(0)
            compute_scatter(0, i0)

            wait_idx(0)
            issue_gat(0, i0 + 2)
            wait_gat(1)
            compute_scatter(1, i0 + 1)
            return 0

        lax.fori_loop(0, CHW // 2, _pair, 0)
        wait_gat(0)
        wait_idx(1)
        wait_sc(0)
        wait_sc(1)
        plsc.subcore_barrier()

        def _out(start, n):
            pltpu.sync_copy(agg_sh.at[pl.ds(start, n)],
                            agg_p.at[c, pl.ds(start, n)])

        _spmem_rows(s, _out, TAIL_ROWS)

    f = pl.kernel(
        body,
        out_type=[
            jax.ShapeDtypeStruct((EALLOC * H,), _F32),
            jax.ShapeDtypeStruct((NC, N, D), _F32),
        ],
        mesh=plsc.VectorSubcoreMesh(core_axis_name="c", subcore_axis_name="s"),
        scratch_types=[
            pltpu.VMEM_SHARED((N + 8, D), _F32),
            pltpu.VMEM((K,), _I32),
            pltpu.VMEM((K,), _I32),
            pltpu.VMEM((K,), _I32),
            pltpu.VMEM((K,), _I32),
            pltpu.VMEM((K,), _I32),
            pltpu.VMEM((K,), _I32),
            pltpu.VMEM((K, D), _F32),
            pltpu.VMEM((K, D), _F32),
            pltpu.VMEM((K, ED), _F32),
            pltpu.VMEM((K, ED), _F32),
            pltpu.VMEM((K * H,), _F32),
            pltpu.VMEM((K * H,), _F32),
            pltpu.VMEM((K * H,), _F32),
            pltpu.VMEM((K * H,), _F32),
            pltpu.VMEM((K, D), _F32),
            pltpu.VMEM((K, D), _F32),
            pltpu.SemaphoreType.DMA,
            pltpu.SemaphoreType.DMA,
            pltpu.SemaphoreType.DMA,
            pltpu.SemaphoreType.DMA,
            pltpu.SemaphoreType.DMA,
            pltpu.SemaphoreType.DMA,
            pltpu.SemaphoreType.DMA,
            pltpu.SemaphoreType.DMA,
        ],
        compiler_params=_SC_PARAMS,
    )
    return f(src_p, dst_p, xl, ex1d, invd16, z128)


def _mid_body(acc_ref, xl_ref, xr_ref, lew_ref, amat_ref,
              inv_ref, anl_ref):
    acc = acc_ref[0] + acc_ref[1]
    asum = acc[:, 16:32]
    cnt = acc[:, 8:9]
    loop_attr = asum / jnp.maximum(cnt, 1.0)
    efl = lax.dot_general(loop_attr, lew_ref[...], (((1,), (1,)), ((), ())),
                          preferred_element_type=_F32)
    t = xl_ref[...] + xr_ref[...] + efl
    t = jnp.where(t > 0, t, t * 0.2)
    alpha = lax.dot_general(t, amat_ref[...], (((1,), (0,)), ((), ())),
                            preferred_element_type=_F32)
    exl = jnp.exp(alpha)
    denom = acc[:, 0:H] + exl
    inv = 1.0 / (denom + 1e-16)
    inv_ref[...] = inv
    anl_ref[...] = exl * inv


def _tc_mid(acc_p, xl, xr, lin_edge_w, amat):
    R = 1000
    row3 = lambda i: (0, i, 0)
    row = lambda i: (i, 0)
    fix = lambda i: (0, 0)
    return pl.pallas_call(
        _mid_body,
        grid=(N // R,),
        in_specs=[
            pl.BlockSpec((NC, R, PW), row3),
            pl.BlockSpec((R, D), row),
            pl.BlockSpec((R, D), row),
            pl.BlockSpec((D, ED), fix),
            pl.BlockSpec((D, H), fix),
        ],
        out_specs=[pl.BlockSpec((R, H), row)] * 2,
        out_shape=[jax.ShapeDtypeStruct((N, H), _F32)] * 2,
    )(acc_p, xl, xr, lin_edge_w, amat)


def _out_body(agg_ref, anl_ref, xl_ref, v_ref, bias_ref, g_ref, b_ref,
              bmat_ref, out_ref):
    anb = lax.dot_general(anl_ref[...], bmat_ref[...], (((1,), (0,)), ((), ())),
                          preferred_element_type=_F32)
    agg = agg_ref[0] + agg_ref[1] + xl_ref[...] * anb + bias_ref[...]
    out_ref[...] = _ln(agg + v_ref[...], g_ref[...], b_ref[...])


def _tc_out(agg_p, anl, xl, v, conv_bias, ln_g, ln_b, bmat):
    R = 1000
    row3 = lambda i: (0, i, 0)
    row = lambda i: (i, 0)
    fix = lambda i: (0, 0)
    return pl.pallas_call(
        _out_body,
        grid=(N // R,),
        in_specs=[
            pl.BlockSpec((NC, R, D), row3),
            pl.BlockSpec((R, H), row),
            pl.BlockSpec((R, D), row),
            pl.BlockSpec((R, D), row),
            pl.BlockSpec((1, D), fix),
            pl.BlockSpec((1, D), fix),
            pl.BlockSpec((1, D), fix),
            pl.BlockSpec((H, D), fix),
        ],
        out_specs=pl.BlockSpec((R, D), row),
        out_shape=jax.ShapeDtypeStruct((N, D), _F32),
    )(agg_p, anl, xl, v, conv_bias.reshape(1, D), ln_g.reshape(1, D),
      ln_b.reshape(1, D), bmat)


def kernel(h, edge_index, edge_attr, w1_w, w1_b, w2_w, w2_b, ln_g, ln_b,
           lin_l_w, lin_l_b, lin_r_w, lin_r_b, lin_edge_w, att, conv_bias):
    src = edge_index[0].astype(_I32)
    dst = edge_index[1].astype(_I32)
    pad = EALLOC - E
    src_p = jnp.concatenate([src, jnp.zeros((pad,), _I32)])
    dst_p = jnp.concatenate([dst, jnp.full((pad,), N, _I32)])
    ea_p = jnp.concatenate([edge_attr, jnp.zeros((pad, ED), _F32)])

    v, xl, xr = _tc_dense(h, w1_w, w1_b, w2_w, w2_b, ln_g, ln_b,
                          lin_l_w, lin_l_b, lin_r_w, lin_r_b)
    xr_p = jnp.concatenate([xr, jnp.zeros((8, D), _F32)])
    ef = _tc_ef(ea_p, lin_edge_w)

    z32 = jnp.zeros((N + 8, PW), _F32)
    z128 = jnp.zeros((N + 8, D), _F32)
    ex1d, acc_p = _sc_pass1(src_p, dst_p, xl, xr_p, ef, ea_p, att, z32)

    amat = jnp.zeros((D, H), _F32).at[
        jnp.arange(D), jnp.arange(D) // C].set(att.reshape(-1))
    invd, anl = _tc_mid(acc_p, xl, xr, lin_edge_w, amat)

    invd16 = jnp.concatenate(
        [jnp.concatenate([invd, jnp.zeros((N, ED - H), _F32)], axis=1),
         jnp.zeros((8, ED), _F32)], axis=0)
    an1d, agg_p = _sc_pass2(src_p, dst_p, xl, ex1d, invd16, z128)
    an_edges = an1d[:E * H].reshape(E, H)

    bmat = jnp.kron(jnp.eye(H, dtype=_F32), jnp.ones((1, C), _F32))
    output = _tc_out(agg_p, anl, xl, v, conv_bias, ln_g, ln_b, bmat)

    alpha_n = jnp.concatenate([an_edges, anl], axis=0)
    return (output, alpha_n)


# speedup vs baseline: 1.1162x; 1.0470x over previous
import jax
import jax.numpy as jnp
from jax import lax
from jax.experimental import pallas as pl
from jax.experimental.pallas import tpu as pltpu
from jax.experimental.pallas import tpu_sc as plsc

N = 10000
E = 320000
D = 128
H = 8
C = 16
HID = 512
ED = 16

NC = 2
NS = 16
NW = NC * NS
K1 = 64
CHW1 = 158
K2 = 64
CHW2 = 158
EALLOC = (CHW1 + 2) * NW * K1
PW = 32
ROWS_PER_SUB = 640
TAIL_ROWS = N - 15 * ROWS_PER_SUB
TAIL_ROWS_Z = N + 8 - 15 * ROWS_PER_SUB

_F32 = jnp.float32
_I32 = jnp.int32

_SC_PARAMS = pltpu.CompilerParams(needs_layout_passes=False,
                                  use_tc_tiling_on_sc=False)


def _ln(r, g, b):
    mu = jnp.mean(r, axis=-1, keepdims=True)
    d = r - mu
    var = jnp.mean(d * d, axis=-1, keepdims=True)
    return d * lax.rsqrt(var + 1e-5) * g + b


def _dense_body(h_ref, w1_ref, w1b_ref, w2_ref, w2b_ref, g_ref, b_ref,
                ll_ref, llb_ref, lr_ref, lrb_ref, v_ref, xl_ref, xr_ref):
    hb = h_ref[...]
    a = lax.dot_general(hb, w1_ref[...], (((1,), (1,)), ((), ())),
                        preferred_element_type=_F32) + w1b_ref[...]
    a = jnp.maximum(a, 0.0)
    u = lax.dot_general(a, w2_ref[...], (((1,), (1,)), ((), ())),
                        preferred_element_type=_F32) + w2b_ref[...]
    v = _ln(hb + u, g_ref[...], b_ref[...])
    v_ref[...] = v
    xl_ref[...] = lax.dot_general(v, ll_ref[...], (((1,), (1,)), ((), ())),
                                  preferred_element_type=_F32) + llb_ref[...]
    xr_ref[...] = lax.dot_general(v, lr_ref[...], (((1,), (1,)), ((), ())),
                                  preferred_element_type=_F32) + lrb_ref[...]


def _tc_dense(h, w1_w, w1_b, w2_w, w2_b, ln_g, ln_b, ll_w, ll_b, lr_w, lr_b):
    R = 1000
    row = lambda i: (i, 0)
    fix = lambda i: (0, 0)
    out = [jax.ShapeDtypeStruct((N, D), _F32)] * 3
    return pl.pallas_call(
        _dense_body,
        grid=(N // R,),
        in_specs=[
            pl.BlockSpec((R, D), row),
            pl.BlockSpec((HID, D), fix),
            pl.BlockSpec((1, HID), fix),
            pl.BlockSpec((D, HID), fix),
            pl.BlockSpec((1, D), fix),
            pl.BlockSpec((1, D), fix),
            pl.BlockSpec((1, D), fix),
            pl.BlockSpec((D, D), fix),
            pl.BlockSpec((1, D), fix),
            pl.BlockSpec((D, D), fix),
            pl.BlockSpec((1, D), fix),
        ],
        out_specs=[pl.BlockSpec((R, D), row)] * 3,
        out_shape=out,
    )(h, w1_w, w1_b.reshape(1, HID), w2_w, w2_b.reshape(1, D),
      ln_g.reshape(1, D), ln_b.reshape(1, D), ll_w, ll_b.reshape(1, D),
      lr_w, lr_b.reshape(1, D))


def _ef_body(ea_ref, w_ref, out_ref):
    out_ref[...] = lax.dot_general(ea_ref[...], w_ref[...],
                                   (((1,), (1,)), ((), ())),
                                   preferred_element_type=_F32)


def _tc_ef(ea_p, lin_edge_w):
    R = 8192
    return pl.pallas_call(
        _ef_body,
        grid=(EALLOC // R,),
        in_specs=[pl.BlockSpec((R, ED), lambda i: (i, 0)),
                  pl.BlockSpec((D, ED), lambda i: (0, 0))],
        out_specs=pl.BlockSpec((R, D), lambda i: (i, 0)),
        out_shape=jax.ShapeDtypeStruct((EALLOC, D), _F32),
    )(ea_p, lin_edge_w)


def _spmem_rows(s, fn, tail):
    @pl.when(s < 15)
    def _():
        fn(s * ROWS_PER_SUB, ROWS_PER_SUB)

    @pl.when(s == 15)
    def _():
        fn(15 * ROWS_PER_SUB, tail)


def _sc_pass1(src_p, dst_p, xl, xr_p, ef_p, ea_p, att, z32):
    K = K1
    CHW = CHW1

    def body(src_h, dst_h, xl_h, xr_h, ef_h, ea_h, att_h, z32_h,
             ex_out, acc_p, xls_out,
             acc_sh,
             sidx0, sidx1, didx0, didx1, dsc0, dsc1, xlb0, xlb1, xrb0, xrb1,
             efb0, efb1, pay0, pay1, exb0, exb1, attbuf,
             semi0, semi1, semg0, semg1, sems0, sems1, semea0, semea1,
             semw0, semw1, semx0, semx1):
        c = lax.axis_index("c")
        s = lax.axis_index("s")
        wid = s * NC + c
        sidx = (sidx0, sidx1)
        didx = (didx0, didx1)
        dsc = (dsc0, dsc1)
        xlb = (xlb0, xlb1)
        xrb = (xrb0, xrb1)
        efb = (efb0, efb1)
        pay = (pay0, pay1)
        exb = (exb0, exb1)
        semi = (semi0, semi1)
        semg = (semg0, semg1)
        sems = (sems0, sems1)
        semea = (semea0, semea1)
        semw = (semw0, semw1)
        semx = (semx0, semx1)

        pltpu.sync_copy(att_h, attbuf)

        zv = jnp.zeros((16,), _F32)
        cntv = jnp.where(lax.iota(_I32, 16) == 8, 1.0, 0.0).astype(_F32)

        def _initrow(e, _):
            for t in range(2):
                pay[t][e, pl.ds(0, 16)] = cntv
                pay[t][e, pl.ds(16, 16)] = zv
            return 0

        lax.fori_loop(0, K, _initrow, 0)

        def _z(start, n):
            pltpu.sync_copy(z32_h.at[pl.ds(start, n)],
                            acc_sh.at[pl.ds(start, n)])

        _spmem_rows(s, _z, TAIL_ROWS_Z)
        plsc.subcore_barrier()

        def _base(i):
            return (i * NW + wid) * K

        def issue_idx(t, i):
            b = _base(i)
            pltpu.async_copy(src_h.at[pl.ds(b, K)], sidx[t], semi[t])
            pltpu.async_copy(dst_h.at[pl.ds(b, K)], didx[t], semi[t])

        def wait_idx(t):
            pltpu.make_async_copy(src_h.at[pl.ds(0, K)], sidx[t], semi[t]).wait()
            pltpu.make_async_copy(dst_h.at[pl.ds(0, K)], didx[t], semi[t]).wait()

        def issue_gat(t, i):
            b = _base(i)
            pltpu.async_copy(xl_h.at[sidx[t]], xlb[t], semg[t])
            pltpu.async_copy(xr_h.at[didx[t]], xrb[t], semg[t])
            pltpu.async_copy(ef_h.at[pl.ds(b, K)], efb[t], semg[t])

        def wait_gat(t):
            pltpu.make_async_copy(xl_h.at[sidx[t]], xlb[t], semg[t]).wait()
            pltpu.make_async_copy(xr_h.at[didx[t]], xrb[t], semg[t]).wait()
            pltpu.make_async_copy(ef_h.at[pl.ds(0, K)], efb[t], semg[t]).wait()

        def wait_sc(t):
            pltpu.make_async_copy(
                exb[t], ex_out.at[pl.ds(0, K * H)], semw[t]).wait()
            pltpu.make_async_copy(
                pay[t], acc_sh.at[dsc[t]], sems[t]).wait()

        def wait_xw(t):
            pltpu.make_async_copy(
                xlb[t], xls_out.at[pl.ds(0, K)], semx[t]).wait()

        def compute_scatter(t, i):
            pltpu.async_copy(xlb[t], xls_out.at[pl.ds(_base(i), K)], semx[t])

            @pl.when(i >= 2)
            def _():
                wait_sc(t)

            pltpu.async_copy(ea_h.at[pl.ds(_base(i), K)],
                             pay[t].at[:, pl.ds(16, ED)], semea[t])

            def _cpidx(q, _):
                dsc[t][pl.ds(q * 16, 16)] = didx[t][pl.ds(q * 16, 16)]
                return 0

            lax.fori_loop(0, K // 16, _cpidx, 0)
            issue_idx(t, i + 2)

            def _group(g, _):
                ev = lax.iota(_I32, 16) + g * 16
                ev8 = ev * H
                for h in range(H):
                    attrow = attbuf[h, :]
                    acc = jnp.zeros((16,), _F32)
                    for cc in range(C):
                        col = h * C + cc
                        cs = jnp.full((16,), col, _I32)
                        tv = (plsc.load_gather(xlb[t], [ev, cs])
                              + plsc.load_gather(xrb[t], [ev, cs])
                              + plsc.load_gather(efb[t], [ev, cs]))
                        tv = jnp.where(tv > 0, tv, tv * 0.2)
                        acc = acc + attrow[cc] * tv
                    exv = jnp.exp(acc)
                    plsc.store_scatter(
                        pay[t], [ev, jnp.full((16,), h, _I32)], exv)
                    plsc.store_scatter(exb[t], [ev8 + h], exv)
                return 0

            lax.fori_loop(0, K // 16, _group, 0)

            pltpu.make_async_copy(ea_h.at[pl.ds(0, K)],
                                  pay[t].at[:, pl.ds(16, ED)],
                                  semea[t]).wait()
            pltpu.async_copy(exb[t], ex_out.at[pl.ds(_base(i) * H, K * H)],
                             semw[t])
            pltpu.async_copy(pay[t], acc_sh.at[dsc[t]], sems[t], add=True)

        pltpu.sync_copy(src_h.at[pl.ds(_base(0), K)], sidx0)
        pltpu.sync_copy(dst_h.at[pl.ds(_base(0), K)], didx0)
        issue_gat(0, 0)
        issue_idx(1, 1)

        def _pair(j, _):
            i0 = 2 * j
            wait_idx(1)

            @pl.when(i0 >= 1)
            def _():
                wait_xw(1)

            issue_gat(1, i0 + 1)
            wait_gat(0)
            compute_scatter(0, i0)

            wait_idx(0)
            wait_xw(0)
            issue_gat(0, i0 + 2)
            wait_gat(1)
            compute_scatter(1, i0 + 1)
            return 0

        lax.fori_loop(0, CHW // 2, _pair, 0)
        wait_gat(0)
        wait_idx(1)
        wait_sc(0)
        wait_sc(1)
        wait_xw(1)
        plsc.subcore_barrier()

        def _out(start, n):
            pltpu.sync_copy(acc_sh.at[pl.ds(start, n)],
                            acc_p.at[c, pl.ds(start, n)])

        _spmem_rows(s, _out, TAIL_ROWS)

    f = pl.kernel(
        body,
        out_type=[
            jax.ShapeDtypeStruct((EALLOC * H,), _F32),
            jax.ShapeDtypeStruct((NC, N, PW), _F32),
            jax.ShapeDtypeStruct((EALLOC, D), _F32),
        ],
        mesh=plsc.VectorSubcoreMesh(core_axis_name="c", subcore_axis_name="s"),
        scratch_types=[
            pltpu.VMEM_SHARED((N + 8, PW), _F32),
            pltpu.VMEM((K,), _I32),
            pltpu.VMEM((K,), _I32),
            pltpu.VMEM((K,), _I32),
            pltpu.VMEM((K,), _I32),
            pltpu.VMEM((K,), _I32),
            pltpu.VMEM((K,), _I32),
            pltpu.VMEM((K, D), _F32),
            pltpu.VMEM((K, D), _F32),
            pltpu.VMEM((K, D), _F32),
            pltpu.VMEM((K, D), _F32),
            pltpu.VMEM((K, D), _F32),
            pltpu.VMEM((K, D), _F32),
            pltpu.VMEM((K, PW), _F32),
            pltpu.VMEM((K, PW), _F32),
            pltpu.VMEM((K * H,), _F32),
            pltpu.VMEM((K * H,), _F32),
            pltpu.VMEM((H, C), _F32),
            pltpu.SemaphoreType.DMA,
            pltpu.SemaphoreType.DMA,
            pltpu.SemaphoreType.DMA,
            pltpu.SemaphoreType.DMA,
            pltpu.SemaphoreType.DMA,
            pltpu.SemaphoreType.DMA,
            pltpu.SemaphoreType.DMA,
            pltpu.SemaphoreType.DMA,
            pltpu.SemaphoreType.DMA,
            pltpu.SemaphoreType.DMA,
            pltpu.SemaphoreType.DMA,
            pltpu.SemaphoreType.DMA,
        ],
        compiler_params=_SC_PARAMS,
    )
    return f(src_p, dst_p, xl, xr_p, ef_p, ea_p, att, z32)


def _sc_pass2(dst_p, xls, ex1d, invd16, z128):
    K = K2
    CHW = CHW2

    def body(dst_h, xls_h, ex_h, invd_h, z128_h,
             an_out, agg_p,
             agg_sh,
             didx0, didx1, dsc0, dsc1, xlb0, xlb1, invb0, invb1,
             exb0, exb1, anb0, anb1, msg0, msg1,
             semi0, semi1, semg0, semg1, sems0, sems1, semw0, semw1):
        c = lax.axis_index("c")
        s = lax.axis_index("s")
        wid = s * NC + c
        didx = (didx0, didx1)
        dsc = (dsc0, dsc1)
        xlb = (xlb0, xlb1)
        invb = (invb0, invb1)
        exb = (exb0, exb1)
        anb = (anb0, anb1)
        msg = (msg0, msg1)
        semi = (semi0, semi1)
        semg = (semg0, semg1)
        sems = (sems0, sems1)
        semw = (semw0, semw1)

        def _z(start, n):
            pltpu.sync_copy(z128_h.at[pl.ds(start, n)],
                            agg_sh.at[pl.ds(start, n)])

        _spmem_rows(s, _z, TAIL_ROWS_Z)
        plsc.subcore_barrier()

        def _base(i):
            return (i * NW + wid) * K

        def issue_idx(t, i):
            b = _base(i)
            pltpu.async_copy(dst_h.at[pl.ds(b, K)], didx[t], semi[t])

        def wait_idx(t):
            pltpu.make_async_copy(dst_h.at[pl.ds(0, K)], didx[t], semi[t]).wait()

        def issue_gat(t, i):
            b = _base(i)
            pltpu.async_copy(xls_h.at[pl.ds(b, K)], xlb[t], semg[t])
            pltpu.async_copy(invd_h.at[didx[t]], invb[t], semg[t])
            pltpu.async_copy(ex_h.at[pl.ds(b * H, K * H)], exb[t], semg[t])

        def wait_gat(t):
            pltpu.make_async_copy(xls_h.at[pl.ds(0, K)], xlb[t], semg[t]).wait()
            pltpu.make_async_copy(invd_h.at[didx[t]], invb[t], semg[t]).wait()
            pltpu.make_async_copy(ex_h.at[pl.ds(0, K * H)], exb[t], semg[t]).wait()

        def wait_sc(t):
            pltpu.make_async_copy(
                anb[t], an_out.at[pl.ds(0, K * H)], semw[t]).wait()
            pltpu.make_async_copy(
                msg[t], agg_sh.at[dsc[t]], sems[t]).wait()

        def compute_scatter(t, i):
            @pl.when(i >= 2)
            def _():
                wait_sc(t)

            def _cpidx(q, _):
                dsc[t][pl.ds(q * 16, 16)] = didx[t][pl.ds(q * 16, 16)]
                return 0

            lax.fori_loop(0, K // 16, _cpidx, 0)
            issue_idx(t, i + 2)

            def _group(g, _):
                ev = lax.iota(_I32, 16) + g * 16
                ev8 = ev * H
                for h in range(H):
                    hs = jnp.full((16,), h, _I32)
                    exv = plsc.load_gather(exb[t], [ev8 + h])
                    iv = plsc.load_gather(invb[t], [ev, hs])
                    an = exv * iv
                    plsc.store_scatter(anb[t], [ev8 + h], an)
                    xvs = [plsc.load_gather(
                        xlb[t], [ev, jnp.full((16,), h * C + cc, _I32)])
                        for cc in range(C)]
                    mvs = [xv * an for xv in xvs]
                    for cc in range(C):
                        cs = jnp.full((16,), h * C + cc, _I32)
                        plsc.store_scatter(msg[t], [ev, cs], mvs[cc])
                return 0

            lax.fori_loop(0, K // 16, _group, 0)

            pltpu.async_copy(anb[t], an_out.at[pl.ds(_base(i) * H, K * H)],
                             semw[t])
            pltpu.async_copy(msg[t], agg_sh.at[dsc[t]], sems[t], add=True)

        pltpu.sync_copy(dst_h.at[pl.ds(_base(0), K)], didx0)
        issue_gat(0, 0)
        issue_idx(1, 1)

        def _pair(j, _):
            i0 = 2 * j
            wait_idx(1)
            issue_gat(1, i0 + 1)
            wait_gat(0)
            compute_scatter(0, i0)

            wait_idx(0)
            issue_gat(0, i0 + 2)
            wait_gat(1)
            compute_scatter(1, i0 + 1)
            return 0

        lax.fori_loop(0, CHW // 2, _pair, 0)
        wait_gat(0)
        wait_idx(1)
        wait_sc(0)
        wait_sc(1)
        plsc.subcore_barrier()

        def _out(start, n):
            pltpu.sync_copy(agg_sh.at[pl.ds(start, n)],
                            agg_p.at[c, pl.ds(start, n)])

        _spmem_rows(s, _out, TAIL_ROWS)

    f = pl.kernel(
        body,
        out_type=[
            jax.ShapeDtypeStruct((EALLOC * H,), _F32),
            jax.ShapeDtypeStruct((NC, N, D), _F32),
        ],
        mesh=plsc.VectorSubcoreMesh(core_axis_name="c", subcore_axis_name="s"),
        scratch_types=[
            pltpu.VMEM_SHARED((N + 8, D), _F32),
            pltpu.VMEM((K,), _I32),
            pltpu.VMEM((K,), _I32),
            pltpu.VMEM((K,), _I32),
            pltpu.VMEM((K,), _I32),
            pltpu.VMEM((K, D), _F32),
            pltpu.VMEM((K, D), _F32),
            pltpu.VMEM((K, ED), _F32),
            pltpu.VMEM((K, ED), _F32),
            pltpu.VMEM((K * H,), _F32),
            pltpu.VMEM((K * H,), _F32),
            pltpu.VMEM((K * H,), _F32),
            pltpu.VMEM((K * H,), _F32),
            pltpu.VMEM((K, D), _F32),
            pltpu.VMEM((K, D), _F32),
            pltpu.SemaphoreType.DMA,
            pltpu.SemaphoreType.DMA,
            pltpu.SemaphoreType.DMA,
            pltpu.SemaphoreType.DMA,
            pltpu.SemaphoreType.DMA,
            pltpu.SemaphoreType.DMA,
            pltpu.SemaphoreType.DMA,
            pltpu.SemaphoreType.DMA,
        ],
        compiler_params=_SC_PARAMS,
    )
    return f(dst_p, xls, ex1d, invd16, z128)


def _mid_body(acc_ref, xl_ref, xr_ref, lew_ref, amat_ref,
              inv_ref, anl_ref):
    acc = acc_ref[0] + acc_ref[1]
    asum = acc[:, 16:32]
    cnt = acc[:, 8:9]
    loop_attr = asum / jnp.maximum(cnt, 1.0)
    efl = lax.dot_general(loop_attr, lew_ref[...], (((1,), (1,)), ((), ())),
                          preferred_element_type=_F32)
    t = xl_ref[...] + xr_ref[...] + efl
    t = jnp.where(t > 0, t, t * 0.2)
    alpha = lax.dot_general(t, amat_ref[...], (((1,), (0,)), ((), ())),
                            preferred_element_type=_F32)
    exl = jnp.exp(alpha)
    denom = acc[:, 0:H] + exl
    inv = 1.0 / (denom + 1e-16)
    inv_ref[...] = inv
    anl_ref[...] = exl * inv


def _tc_mid(acc_p, xl, xr, lin_edge_w, amat):
    R = 1000
    row3 = lambda i: (0, i, 0)
    row = lambda i: (i, 0)
    fix = lambda i: (0, 0)
    return pl.pallas_call(
        _mid_body,
        grid=(N // R,),
        in_specs=[
            pl.BlockSpec((NC, R, PW), row3),
            pl.BlockSpec((R, D), row),
            pl.BlockSpec((R, D), row),
            pl.BlockSpec((D, ED), fix),
            pl.BlockSpec((D, H), fix),
        ],
        out_specs=[pl.BlockSpec((R, H), row)] * 2,
        out_shape=[jax.ShapeDtypeStruct((N, H), _F32)] * 2,
    )(acc_p, xl, xr, lin_edge_w, amat)


def _out_body(agg_ref, anl_ref, xl_ref, v_ref, bias_ref, g_ref, b_ref,
              bmat_ref, out_ref):
    anb = lax.dot_general(anl_ref[...], bmat_ref[...], (((1,), (0,)), ((), ())),
                          preferred_element_type=_F32)
    agg = agg_ref[0] + agg_ref[1] + xl_ref[...] * anb + bias_ref[...]
    out_ref[...] = _ln(agg + v_ref[...], g_ref[...], b_ref[...])


def _tc_out(agg_p, anl, xl, v, conv_bias, ln_g, ln_b, bmat):
    R = 1000
    row3 = lambda i: (0, i, 0)
    row = lambda i: (i, 0)
    fix = lambda i: (0, 0)
    return pl.pallas_call(
        _out_body,
        grid=(N // R,),
        in_specs=[
            pl.BlockSpec((NC, R, D), row3),
            pl.BlockSpec((R, H), row),
            pl.BlockSpec((R, D), row),
            pl.BlockSpec((R, D), row),
            pl.BlockSpec((1, D), fix),
            pl.BlockSpec((1, D), fix),
            pl.BlockSpec((1, D), fix),
            pl.BlockSpec((H, D), fix),
        ],
        out_specs=pl.BlockSpec((R, D), row),
        out_shape=jax.ShapeDtypeStruct((N, D), _F32),
    )(agg_p, anl, xl, v, conv_bias.reshape(1, D), ln_g.reshape(1, D),
      ln_b.reshape(1, D), bmat)


def kernel(h, edge_index, edge_attr, w1_w, w1_b, w2_w, w2_b, ln_g, ln_b,
           lin_l_w, lin_l_b, lin_r_w, lin_r_b, lin_edge_w, att, conv_bias):
    src = edge_index[0].astype(_I32)
    dst = edge_index[1].astype(_I32)
    pad = EALLOC - E
    src_p = jnp.concatenate([src, jnp.zeros((pad,), _I32)])
    dst_p = jnp.concatenate([dst, jnp.full((pad,), N, _I32)])
    ea_p = jnp.concatenate([edge_attr, jnp.zeros((pad, ED), _F32)])

    v, xl, xr = _tc_dense(h, w1_w, w1_b, w2_w, w2_b, ln_g, ln_b,
                          lin_l_w, lin_l_b, lin_r_w, lin_r_b)
    xr_p = jnp.concatenate([xr, jnp.zeros((8, D), _F32)])
    ef = _tc_ef(ea_p, lin_edge_w)

    z32 = jnp.zeros((N + 8, PW), _F32)
    z128 = jnp.zeros((N + 8, D), _F32)
    ex1d, acc_p, xls = _sc_pass1(src_p, dst_p, xl, xr_p, ef, ea_p, att, z32)

    amat = jnp.zeros((D, H), _F32).at[
        jnp.arange(D), jnp.arange(D) // C].set(att.reshape(-1))
    invd, anl = _tc_mid(acc_p, xl, xr, lin_edge_w, amat)

    invd16 = jnp.concatenate(
        [jnp.concatenate([invd, jnp.zeros((N, ED - H), _F32)], axis=1),
         jnp.zeros((8, ED), _F32)], axis=0)
    an1d, agg_p = _sc_pass2(dst_p, xls, ex1d, invd16, z128)
    an_edges = an1d[:E * H].reshape(E, H)

    bmat = jnp.kron(jnp.eye(H, dtype=_F32), jnp.ones((1, C), _F32))
    output = _tc_out(agg_p, anl, xl, v, conv_bias, ln_g, ln_b, bmat)

    alpha_n = jnp.concatenate([an_edges, anl], axis=0)
    return (output, alpha_n)
